# baseline (speedup 1.0000x reference)
import functools

import jax
import jax.numpy as jnp
from jax import lax
from jax.experimental import pallas as pl
from jax.experimental.pallas import tpu as pltpu
from jax.experimental.pallas import tpu_sc as plsc


@functools.lru_cache(maxsize=None)
def _make_gather(n_total: int, vocab: int, hidden: int):
    info = plsc.get_sparse_core_info()
    num_cores, num_subcores = info.num_cores, info.num_subcores
    num_workers = num_cores * num_subcores
    assert n_total % num_workers == 0
    n_per_w = n_total // num_workers
    chunk = 16
    nbuf = 6
    assert n_per_w % chunk == 0
    n_chunks = n_per_w // chunk

    mesh = plsc.VectorSubcoreMesh(core_axis_name="c", subcore_axis_name="s")

    @functools.partial(
        pl.kernel,
        mesh=mesh,
        out_type=jax.ShapeDtypeStruct((n_total, hidden), jnp.float32),
        scratch_types=[
            pltpu.VMEM((n_per_w,), jnp.int32),
        ]
        + [pltpu.VMEM((chunk, hidden), jnp.float32) for _ in range(nbuf)]
        + [pltpu.SemaphoreType.DMA for _ in range(2 * nbuf)],
    )
    def gather_kernel(table_hbm, idx_hbm, out_hbm, idx_v, *scratch):
        bufs = scratch[:nbuf]
        gsems = scratch[nbuf : 2 * nbuf]
        ssems = scratch[2 * nbuf :]
        wid = lax.axis_index("s") * num_cores + lax.axis_index("c")
        base = wid * n_per_w
        pltpu.sync_copy(idx_hbm.at[pl.ds(base, n_per_w)], idx_v)
        g_h = [None] * n_chunks
        s_h = [None] * n_chunks
        for c in range(n_chunks + 1):
            if c < n_chunks:
                i = c % nbuf
                if c >= nbuf:
                    s_h[c - nbuf].wait()
                g_h[c] = pltpu.async_copy(
                    table_hbm.at[idx_v.at[pl.ds(c * chunk, chunk)]],
                    bufs[i], gsems[i],
                )
            d = c - 1
            if d >= 0:
                g_h[d].wait()
                s_h[d] = pltpu.async_copy(
                    bufs[d % nbuf],
                    out_hbm.at[pl.ds(base + d * chunk, chunk)],
                    ssems[d % nbuf],
                )
        for d in range(max(0, n_chunks - nbuf), n_chunks):
            s_h[d].wait()

    return gather_kernel


def kernel(input_ids, embed_tokens):
    b, s = input_ids.shape
    vocab, hidden = embed_tokens.shape
    flat_ids = input_ids.reshape(-1).astype(jnp.int32)
    gather = _make_gather(b * s, vocab, hidden)
    out = gather(embed_tokens, flat_ids)
    return out.reshape(b, s, hidden)


# speedup vs baseline: 1.0276x; 1.0276x over previous
import functools

import jax
import jax.numpy as jnp
from jax import lax
from jax.experimental import pallas as pl
from jax.experimental.pallas import tpu as pltpu
from jax.experimental.pallas import tpu_sc as plsc


@functools.lru_cache(maxsize=None)
def _make_gather(n_total: int, vocab: int, hidden: int):
    info = plsc.get_sparse_core_info()
    num_cores, num_subcores = info.num_cores, info.num_subcores
    num_workers = num_cores * num_subcores
    assert n_total % num_workers == 0
    n_per_w = n_total // num_workers
    chunk = 16
    nbuf = 6
    assert n_per_w % chunk == 0
    n_chunks = n_per_w // chunk

    mesh = plsc.VectorSubcoreMesh(core_axis_name="c", subcore_axis_name="s")

    @functools.partial(
        pl.kernel,
        mesh=mesh,
        out_type=jax.ShapeDtypeStruct((n_total, hidden), jnp.float32),
        scratch_types=[
            pltpu.VMEM((n_per_w,), jnp.int32),
        ]
        + [pltpu.VMEM((chunk, hidden), jnp.float32) for _ in range(nbuf)]
        + [pltpu.SemaphoreType.DMA for _ in range(2 * nbuf)],
    )
    def gather_kernel(table_hbm, idx_hbm, out_hbm, idx_v, *scratch):
        bufs = scratch[:nbuf]
        gsems = scratch[nbuf : 2 * nbuf]
        ssems = scratch[2 * nbuf :]
        wid = lax.axis_index("s") * num_cores + lax.axis_index("c")
        base = wid * n_per_w
        pltpu.sync_copy(idx_hbm.at[pl.ds(base, n_per_w)], idx_v)
        la = 4
        g_h = [None] * n_chunks
        s_h = [None] * n_chunks
        for c in range(n_chunks + la):
            if c < n_chunks:
                i = c % nbuf
                if c >= nbuf:
                    s_h[c - nbuf].wait()
                g_h[c] = pltpu.async_copy(
                    table_hbm.at[idx_v.at[pl.ds(c * chunk, chunk)]],
                    bufs[i], gsems[i],
                )
            d = c - la
            if d >= 0:
                g_h[d].wait()
                s_h[d] = pltpu.async_copy(
                    bufs[d % nbuf],
                    out_hbm.at[pl.ds(base + d * chunk, chunk)],
                    ssems[d % nbuf],
                )
        for d in range(max(0, n_chunks - nbuf), n_chunks):
            s_h[d].wait()

    return gather_kernel


def kernel(input_ids, embed_tokens):
    b, s = input_ids.shape
    vocab, hidden = embed_tokens.shape
    flat_ids = input_ids.reshape(-1).astype(jnp.int32)
    gather = _make_gather(b * s, vocab, hidden)
    out = gather(embed_tokens, flat_ids)
    return out.reshape(b, s, hidden)
